# baseline (speedup 1.0000x reference)
import jax
import jax.numpy as jnp
from jax import lax
from jax.experimental import pallas as pl
from jax.experimental.pallas import tpu as pltpu
from jax.experimental.pallas import tpu_sc as plsc

RB = 16


def _argmin_kernel(xb_ref, et_ref, idx_ref, flat_ref):
    cols = []
    fcols = []
    for c in range(4):
        xc = xb_ref[0, :, c, :]

        def dk(k):
            d = xc[:, k:k + 1] - et_ref[c, k]
            return d * d

        d2 = None
        for j in range(8):
            b = 8 * j
            p0 = dk(b + 0) + dk(b + 4)
            p2 = dk(b + 2) + dk(b + 6)
            p1 = dk(b + 1) + dk(b + 5)
            p3 = dk(b + 3) + dk(b + 7)
            v = (p0 + p2) + (p1 + p3)
            d2 = v if d2 is None else d2 + v
        s = jnp.sqrt(d2)
        mn = jnp.min(s, axis=1, keepdims=True)
        iota = jax.lax.broadcasted_iota(jnp.int32, s.shape, 1)
        idx = jnp.min(jnp.where(s == mn, iota, 1024), axis=1)
        cols.append(idx[:, None])
        fcols.append(idx[:, None] + (c * 1024))
    idx_ref[...] = jnp.concatenate(cols, axis=1)
    flat_ref[...] = jnp.concatenate(fcols, axis=1)


def _sc_gather(table, flat_idx):
    info = plsc.get_sparse_core_info()
    nw = info.num_cores * info.num_subcores
    n = flat_idx.shape[0]
    b_per_w = n // nw
    mesh = plsc.VectorSubcoreMesh(core_axis_name="c", subcore_axis_name="s")

    def k(table_hbm, idx_hbm, out_hbm, idx_v, rows_v, sem):
        wid = lax.axis_index("s") * info.num_cores + lax.axis_index("c")
        base = wid * b_per_w
        pltpu.sync_copy(idx_hbm.at[pl.ds(base, b_per_w)], idx_v)
        pltpu.async_copy(table_hbm.at[idx_v], rows_v, sem).wait()
        pltpu.sync_copy(rows_v, out_hbm.at[pl.ds(base, b_per_w)])

    return pl.kernel(
        k,
        mesh=mesh,
        out_type=jax.ShapeDtypeStruct((n, table.shape[1]), table.dtype),
        scratch_types=[
            pltpu.VMEM((b_per_w,), jnp.int32),
            pltpu.VMEM((b_per_w, table.shape[1]), table.dtype),
            pltpu.SemaphoreType.DMA,
        ],
    )(table, flat_idx)


def _assemble_kernel(idxf_ref, g_ref, x_ref, out0_ref, out1_ref, ent_ref):
    iota = jax.lax.broadcasted_iota(jnp.int32, (8192, 1024), 1)
    onehot = (iota == idxf_ref[...]).astype(jnp.float32)
    hist = jnp.sum(onehot, axis=0, keepdims=True)
    g = g_ref[...]
    x = x_ref[...]
    out0_ref[...] = (g - x) + x
    t = x - g
    out1_ref[...] = jnp.sum(t * t, axis=1, keepdims=True)
    p = hist * jnp.float32(1.0 / 2048.0)
    pos = hist > 0
    safe = jnp.where(pos, p, jnp.float32(1.0))
    ent = -jnp.sum(jnp.where(pos, p * jnp.log(safe), jnp.float32(0.0)))
    ent_ref[...] = ent.reshape(1, 1)


def kernel(x0, embedding0):
    xb = x0.reshape(2048 // RB, RB, 4, 64)
    et = embedding0.transpose(0, 2, 1)
    etb = jnp.broadcast_to(et[:, :, None, :], (4, 64, RB, 1024))

    idx, flat = pl.pallas_call(
        _argmin_kernel,
        grid=(2048 // RB,),
        in_specs=[
            pl.BlockSpec((1, RB, 4, 64), lambda i: (i, 0, 0, 0)),
            pl.BlockSpec((4, 64, RB, 1024), lambda i: (0, 0, 0, 0)),
        ],
        out_specs=[
            pl.BlockSpec((RB, 4), lambda i: (i, 0)),
            pl.BlockSpec((RB, 4), lambda i: (i, 0)),
        ],
        out_shape=[
            jax.ShapeDtypeStruct((2048, 4), jnp.int32),
            jax.ShapeDtypeStruct((2048, 4), jnp.int32),
        ],
    )(xb, etb)

    table = jnp.pad(embedding0.reshape(4096, 64), ((0, 0), (0, 64)))
    g = _sc_gather(table, flat.reshape(8192))[:, :64]

    xf = x0.reshape(8192, 64)
    out0f, out1f, ent = pl.pallas_call(
        _assemble_kernel,
        grid=(1,),
        in_specs=[
            pl.BlockSpec((8192, 1), lambda i: (0, 0)),
            pl.BlockSpec((8192, 64), lambda i: (0, 0)),
            pl.BlockSpec((8192, 64), lambda i: (0, 0)),
        ],
        out_specs=[
            pl.BlockSpec((8192, 64), lambda i: (0, 0)),
            pl.BlockSpec((8192, 1), lambda i: (0, 0)),
            pl.BlockSpec((1, 1), lambda i: (0, 0)),
        ],
        out_shape=[
            jax.ShapeDtypeStruct((8192, 64), jnp.float32),
            jax.ShapeDtypeStruct((8192, 1), jnp.float32),
            jax.ShapeDtypeStruct((1, 1), jnp.float32),
        ],
    )(idx.reshape(8192, 1), g, xf)

    out0 = out0f.reshape(4, 512, 4, 64)
    out1 = out1f.reshape(4, 512, 4)
    entropy = ent.reshape(())
    return (out0, out1, out1, entropy)
